# baseline (speedup 1.0000x reference)
import functools

import jax
import jax.numpy as jnp
from jax import lax
from jax.experimental import pallas as pl
from jax.experimental.pallas import tpu as pltpu
from jax.experimental.pallas import tpu_sc as plsc

N = 10000
E = 320000
D = 128
LANES = 16
NC = 2
NS = 16
NW = NC * NS
EPW = E // NW
C = 80
NCHUNK = EPW // C
BI = 5
NBULK = NCHUNK // BI
STRIPE = 640
TAIL = N - 15 * STRIPE


def _sc_edge_body(a_hbm, b_hbm, src_hbm, rel_hbm, dst_hbm,
                  aggp_hbm, degp_hbm,
                  agg_sh, srcb, relb, dstb, dstv, rows_a, rows_b,
                  degv, sem_a, sem_b, sem_s, sem_r, sem_d):
    c = lax.axis_index("c")
    s = lax.axis_index("s")
    w = c * NS + s

    def fill_zrow(i, carry):
        for j in range(D // LANES):
            rows_a[i, pl.ds(j * LANES, LANES)] = jnp.zeros((LANES,),
                                                           jnp.float32)
        return carry
    lax.fori_loop(0, C, fill_zrow, 0)

    def zero_deg(i, carry):
        degv[pl.ds(i * LANES, LANES)] = jnp.zeros((LANES,), jnp.float32)
        return carry
    lax.fori_loop(0, N // LANES, zero_deg, 0)

    base = s * STRIPE
    nz = lax.select(s < 15, STRIPE // C, TAIL // C)

    def zero_stripe(k, carry):
        pltpu.sync_copy(rows_a, agg_sh.at[pl.ds(base + k * C, C)])
        return carry
    lax.fori_loop(0, nz, zero_stripe, 0)
    plsc.subcore_barrier()

    lane = lax.iota(jnp.int32, LANES)
    one = jnp.ones((LANES,), jnp.float32)

    def bulk(u, carry):
        base_e = w * EPW + u * BI * C
        cp_s = pltpu.async_copy(src_hbm.at[pl.ds(base_e, BI * C)],
                                srcb, sem_s)
        cp_r = pltpu.async_copy(rel_hbm.at[pl.ds(base_e, BI * C)],
                                relb, sem_r)
        cp_d = pltpu.async_copy(dst_hbm.at[pl.ds(base_e, BI * C)],
                                dstb, sem_d)
        cp_s.wait()
        cp_r.wait()
        cp_d.wait()

        def chunk(j, carry2):
            cp_a = pltpu.async_copy(a_hbm.at[srcb.at[pl.ds(j * C, C)]],
                                    rows_a, sem_a)
            cp_b = pltpu.async_copy(b_hbm.at[relb.at[pl.ds(j * C, C)]],
                                    rows_b, sem_b)

            for k in range(C // LANES):
                d16 = dstb[pl.ds(j * C + k * LANES, LANES)]
                dstv[pl.ds(k * LANES, LANES)] = d16
                for l in range(LANES):
                    plsc.addupdate_scatter(degv, [d16], one, mask=lane == l)

            cp_a.wait()
            cp_b.wait()

            @plsc.parallel_loop(0, C, unroll=4)
            def edge(e):
                for jj in range(D // LANES):
                    sl = pl.ds(jj * LANES, LANES)
                    v = rows_a[e, sl] + rows_b[e, sl]
                    rows_a[e, sl] = jnp.maximum(v, 0.0)

            pltpu.sync_copy(rows_a, agg_sh.at[dstv], add=True)
            return carry2
        lax.fori_loop(0, BI, chunk, 0)
        return carry
    lax.fori_loop(0, NBULK, bulk, 0)

    plsc.subcore_barrier()

    @pl.when(s < 15)
    def _():
        pltpu.sync_copy(agg_sh.at[pl.ds(base, STRIPE)],
                        aggp_hbm.at[c, pl.ds(base, STRIPE)])

    @pl.when(s == 15)
    def _():
        pltpu.sync_copy(agg_sh.at[pl.ds(15 * STRIPE, TAIL)],
                        aggp_hbm.at[c, pl.ds(15 * STRIPE, TAIL)])

    pltpu.sync_copy(degv, degp_hbm.at[pl.ds(w * N, N)])


_sc_edge = functools.partial(
    pl.kernel,
    out_type=[jax.ShapeDtypeStruct((NC, N, D), jnp.float32),
              jax.ShapeDtypeStruct((NW * N,), jnp.float32)],
    mesh=plsc.VectorSubcoreMesh(core_axis_name="c", subcore_axis_name="s"),
    compiler_params=pltpu.CompilerParams(needs_layout_passes=False),
    scratch_types=[
        pltpu.VMEM_SHARED((N, D), jnp.float32),
        pltpu.VMEM((BI * C,), jnp.int32),
        pltpu.VMEM((BI * C,), jnp.int32),
        pltpu.VMEM((BI * C,), jnp.int32),
        pltpu.VMEM((C,), jnp.int32),
        pltpu.VMEM((C, D), jnp.float32),
        pltpu.VMEM((C, D), jnp.float32),
        pltpu.VMEM((N,), jnp.float32),
        pltpu.SemaphoreType.DMA,
        pltpu.SemaphoreType.DMA,
        pltpu.SemaphoreType.DMA,
        pltpu.SemaphoreType.DMA,
        pltpu.SemaphoreType.DMA,
    ],
)(_sc_edge_body)


def _mm_body(x_ref, w_ref, o_ref):
    o_ref[...] = jnp.dot(x_ref[...], w_ref[...],
                         preferred_element_type=jnp.float32)


def _matmul(x, w, block_rows):
    m, k = x.shape
    _, n = w.shape
    return pl.pallas_call(
        _mm_body,
        grid=(m // block_rows,),
        in_specs=[pl.BlockSpec((block_rows, k), lambda i: (i, 0)),
                  pl.BlockSpec((k, n), lambda i: (0, 0))],
        out_specs=pl.BlockSpec((block_rows, n), lambda i: (i, 0)),
        out_shape=jax.ShapeDtypeStruct((m, n), jnp.float32),
    )(x, w)


def _out_body(ent_ref, aggp_ref, degp_ref, w1_ref, w2_ref, o_ref):
    agg = aggp_ref[0] + aggp_ref[1]
    deg = jnp.sum(degp_ref[...], axis=1, keepdims=True)
    aggn = agg / jnp.maximum(deg, 1.0)
    h = jnp.dot(ent_ref[...], w1_ref[...], preferred_element_type=jnp.float32)
    h = h + jnp.dot(aggn, w2_ref[...], preferred_element_type=jnp.float32)
    o_ref[...] = jnp.maximum(h, 0.0)


def _node_update(ent, aggp, degp, w1, w2, block_rows):
    m = ent.shape[0]
    return pl.pallas_call(
        _out_body,
        grid=(m // block_rows,),
        in_specs=[
            pl.BlockSpec((block_rows, D), lambda i: (i, 0)),
            pl.BlockSpec((NC, block_rows, D), lambda i: (0, i, 0)),
            pl.BlockSpec((block_rows, NW), lambda i: (i, 0)),
            pl.BlockSpec((D, D), lambda i: (0, 0)),
            pl.BlockSpec((D, D), lambda i: (0, 0)),
        ],
        out_specs=pl.BlockSpec((block_rows, D), lambda i: (i, 0)),
        out_shape=jax.ShapeDtypeStruct((m, D), jnp.float32),
    )(ent, aggp, degp, w1, w2)


def kernel(ent_embeds, rel_embeds, W_msg, W_out, edge_index, edge_rel):
    src = edge_index[0]
    dst = edge_index[1]
    a_tab = _matmul(ent_embeds, W_msg[:D], 1000)
    b_tab = _matmul(rel_embeds, W_msg[D:], 256)
    aggp, degflat = _sc_edge(a_tab, b_tab, src, edge_rel, dst)
    degp = degflat.reshape(NW, N).T
    return _node_update(ent_embeds, aggp, degp, W_out[:D], W_out[D:], 1000)


# speedup vs baseline: 1.3254x; 1.3254x over previous
import functools

import jax
import jax.numpy as jnp
from jax import lax
from jax.experimental import pallas as pl
from jax.experimental.pallas import tpu as pltpu
from jax.experimental.pallas import tpu_sc as plsc

N = 10000
E = 320000
D = 128
LANES = 16
NC = 2
NS = 16
NW = NC * NS
EPW = E // NW
C = 80
NCHUNK = EPW // C
BI = 5
NBULK = NCHUNK // BI
STRIPE = 640
TAIL = N - 15 * STRIPE


def _sc_edge_body(a_hbm, b_hbm, src_hbm, rel_hbm, dst_hbm,
                  aggp_hbm, degp_hbm,
                  agg_sh, b_sh, srcb, relb, dstb, dstv, rows_a, rows_b,
                  degv, sem_a, sem_b, sem_s, sem_r, sem_d):
    c = lax.axis_index("c")
    s = lax.axis_index("s")
    w = c * NS + s

    def fill_zrow(i, carry):
        for j in range(D // LANES):
            rows_a[0, i, pl.ds(j * LANES, LANES)] = jnp.zeros((LANES,),
                                                              jnp.float32)
        return carry
    lax.fori_loop(0, C, fill_zrow, 0)

    def zero_deg(i, carry):
        degv[pl.ds(i * LANES, LANES)] = jnp.zeros((LANES,), jnp.float32)
        return carry
    lax.fori_loop(0, N // LANES, zero_deg, 0)

    base = s * STRIPE
    nz = lax.select(s < 15, STRIPE // C, TAIL // C)

    def zero_stripe(k, carry):
        pltpu.sync_copy(rows_a.at[0], agg_sh.at[pl.ds(base + k * C, C)])
        return carry
    lax.fori_loop(0, nz, zero_stripe, 0)

    @pl.when(s == 0)
    def _():
        pltpu.sync_copy(b_hbm, b_sh)
    plsc.subcore_barrier()

    lane = lax.iota(jnp.int32, LANES)
    one = jnp.ones((LANES,), jnp.float32)

    def load_bulk(u):
        base_e = w * EPW + u * BI * C
        cp_s = pltpu.async_copy(src_hbm.at[pl.ds(base_e, BI * C)],
                                srcb, sem_s)
        cp_r = pltpu.async_copy(rel_hbm.at[pl.ds(base_e, BI * C)],
                                relb, sem_r)
        cp_d = pltpu.async_copy(dst_hbm.at[pl.ds(base_e, BI * C)],
                                dstb, sem_d)
        cp_s.wait()
        cp_r.wait()
        cp_d.wait()

    def hist_chunk(off):
        for k in range(C // LANES):
            d16 = dstb[pl.ds(off + k * LANES, LANES)]
            dstv[pl.ds(k * LANES, LANES)] = d16
            for l in range(LANES):
                plsc.addupdate_scatter(degv, [d16], one, mask=lane == l)

    def compute_chunk(slot):
        @plsc.parallel_loop(0, C, unroll=4)
        def edge(e):
            for jj in range(D // LANES):
                sl = pl.ds(jj * LANES, LANES)
                v = rows_a[slot, e, sl] + rows_b[e, sl]
                rows_a[slot, e, sl] = jnp.maximum(v, 0.0)

    load_bulk(0)
    pltpu.async_copy(a_hbm.at[srcb.at[pl.ds(0, C)]], rows_a.at[0],
                     sem_a.at[0]).wait()
    pltpu.async_copy(b_sh.at[relb.at[pl.ds(0, C)]], rows_b, sem_b).wait()

    def chunk(g, carry):
        slot = lax.rem(g, 2)
        nslot = 1 - slot
        off = lax.rem(g, BI) * C
        noff = lax.rem(g + 1, BI) * C

        hist_chunk(off)

        @pl.when(lax.rem(g + 1, BI) == 0)
        def _():
            load_bulk((g + 1) // BI)

        cp_a = pltpu.async_copy(a_hbm.at[srcb.at[pl.ds(noff, C)]],
                                rows_a.at[nslot], sem_a.at[nslot])
        compute_chunk(slot)
        cp_b = pltpu.async_copy(b_sh.at[relb.at[pl.ds(noff, C)]],
                                rows_b, sem_b)
        pltpu.sync_copy(rows_a.at[slot], agg_sh.at[dstv], add=True)
        cp_a.wait()
        cp_b.wait()
        return carry
    lax.fori_loop(0, NCHUNK - 1, chunk, 0)

    hist_chunk(lax.rem(NCHUNK - 1, BI) * C)
    compute_chunk(lax.rem(NCHUNK - 1, 2))
    pltpu.sync_copy(rows_a.at[lax.rem(NCHUNK - 1, 2)], agg_sh.at[dstv],
                    add=True)

    plsc.subcore_barrier()

    @pl.when(s < 15)
    def _():
        pltpu.sync_copy(agg_sh.at[pl.ds(base, STRIPE)],
                        aggp_hbm.at[c, pl.ds(base, STRIPE)])

    @pl.when(s == 15)
    def _():
        pltpu.sync_copy(agg_sh.at[pl.ds(15 * STRIPE, TAIL)],
                        aggp_hbm.at[c, pl.ds(15 * STRIPE, TAIL)])

    pltpu.sync_copy(degv, degp_hbm.at[pl.ds(w * N, N)])


_sc_edge = functools.partial(
    pl.kernel,
    out_type=[jax.ShapeDtypeStruct((NC, N, D), jnp.float32),
              jax.ShapeDtypeStruct((NW * N,), jnp.float32)],
    mesh=plsc.VectorSubcoreMesh(core_axis_name="c", subcore_axis_name="s"),
    compiler_params=pltpu.CompilerParams(needs_layout_passes=False),
    scratch_types=[
        pltpu.VMEM_SHARED((N, D), jnp.float32),
        pltpu.VMEM_SHARED((256, D), jnp.float32),
        pltpu.VMEM((BI * C,), jnp.int32),
        pltpu.VMEM((BI * C,), jnp.int32),
        pltpu.VMEM((BI * C,), jnp.int32),
        pltpu.VMEM((C,), jnp.int32),
        pltpu.VMEM((2, C, D), jnp.float32),
        pltpu.VMEM((C, D), jnp.float32),
        pltpu.VMEM((N,), jnp.float32),
        pltpu.SemaphoreType.DMA((2,)),
        pltpu.SemaphoreType.DMA,
        pltpu.SemaphoreType.DMA,
        pltpu.SemaphoreType.DMA,
        pltpu.SemaphoreType.DMA,
    ],
)(_sc_edge_body)


def _mm_body(x_ref, w_ref, o_ref):
    o_ref[...] = jnp.dot(x_ref[...], w_ref[...],
                         preferred_element_type=jnp.float32)


def _matmul(x, w, block_rows):
    m, k = x.shape
    _, n = w.shape
    return pl.pallas_call(
        _mm_body,
        grid=(m // block_rows,),
        in_specs=[pl.BlockSpec((block_rows, k), lambda i: (i, 0)),
                  pl.BlockSpec((k, n), lambda i: (0, 0))],
        out_specs=pl.BlockSpec((block_rows, n), lambda i: (i, 0)),
        out_shape=jax.ShapeDtypeStruct((m, n), jnp.float32),
    )(x, w)


def _out_body(ent_ref, aggp_ref, degp_ref, w1_ref, w2_ref, o_ref):
    agg = aggp_ref[0] + aggp_ref[1]
    deg = jnp.sum(degp_ref[...], axis=1, keepdims=True)
    aggn = agg / jnp.maximum(deg, 1.0)
    h = jnp.dot(ent_ref[...], w1_ref[...], preferred_element_type=jnp.float32)
    h = h + jnp.dot(aggn, w2_ref[...], preferred_element_type=jnp.float32)
    o_ref[...] = jnp.maximum(h, 0.0)


def _node_update(ent, aggp, degp, w1, w2, block_rows):
    m = ent.shape[0]
    return pl.pallas_call(
        _out_body,
        grid=(m // block_rows,),
        in_specs=[
            pl.BlockSpec((block_rows, D), lambda i: (i, 0)),
            pl.BlockSpec((NC, block_rows, D), lambda i: (0, i, 0)),
            pl.BlockSpec((block_rows, NW), lambda i: (i, 0)),
            pl.BlockSpec((D, D), lambda i: (0, 0)),
            pl.BlockSpec((D, D), lambda i: (0, 0)),
        ],
        out_specs=pl.BlockSpec((block_rows, D), lambda i: (i, 0)),
        out_shape=jax.ShapeDtypeStruct((m, D), jnp.float32),
    )(ent, aggp, degp, w1, w2)


def kernel(ent_embeds, rel_embeds, W_msg, W_out, edge_index, edge_rel):
    src = edge_index[0]
    dst = edge_index[1]
    a_tab = _matmul(ent_embeds, W_msg[:D], 1000)
    b_tab = _matmul(rel_embeds, W_msg[D:], 256)
    aggp, degflat = _sc_edge(a_tab, b_tab, src, edge_rel, dst)
    degp = degflat.reshape(NW, N).T
    return _node_update(ent_embeds, aggp, degp, W_out[:D], W_out[D:], 1000)
